# baseline (speedup 1.0000x reference)
import jax
import jax.numpy as jnp
from jax.experimental import pallas as pl
from jax.experimental.pallas import tpu as pltpu

_K = 16
_SHIFT = _K - 1


def _span_dma_kernel(x_hbm, out_hbm, zeros_vmem, sem):
    B, L, _, _ = x_hbm.shape
    zeros_vmem[...] = jnp.zeros_like(zeros_vmem)
    copies = []
    for b in range(B):
        copies.append(
            pltpu.make_async_copy(x_hbm.at[b], out_hbm.at[b, :, 0], sem)
        )
        copies.append(
            pltpu.make_async_copy(
                x_hbm.at[b, pl.ds(_SHIFT, L - _SHIFT)],
                out_hbm.at[b, pl.ds(0, L - _SHIFT), 1],
                sem,
            )
        )
        copies.append(
            pltpu.make_async_copy(
                zeros_vmem.at[pl.ds(0, _SHIFT)],
                out_hbm.at[b, pl.ds(L - _SHIFT, _SHIFT), 1],
                sem,
            )
        )
    for c in copies:
        c.start()
    for c in copies:
        c.wait()


def kernel(x):
    B, L, D = x.shape
    x4 = x.reshape(B, L, 8, D // 8)
    out = pl.pallas_call(
        _span_dma_kernel,
        in_specs=[pl.BlockSpec(memory_space=pltpu.MemorySpace.HBM)],
        out_specs=pl.BlockSpec(memory_space=pltpu.MemorySpace.HBM),
        out_shape=jax.ShapeDtypeStruct((B, L, 2, 8, D // 8), x.dtype),
        scratch_shapes=[
            pltpu.VMEM((_K, 8, D // 8), x.dtype),
            pltpu.SemaphoreType.DMA,
        ],
    )(x4)
    return out.reshape(B, L, 2, D)


# speedup vs baseline: 64.4239x; 64.4239x over previous
import jax
import jax.numpy as jnp
from jax.experimental import pallas as pl

_K = 16
_SHIFT = _K - 1


def _span_kernel(x_cur_ref, x_nxt_ref, out_ref, *, tl, L):
    i = pl.program_id(1)
    cur = x_cur_ref[0]
    nxt = x_nxt_ref[0]
    shifted = jnp.concatenate([cur[_SHIFT:, :], nxt[:_SHIFT, :]], axis=0)
    row = jax.lax.broadcasted_iota(jnp.int32, shifted.shape, 0)
    g = i * tl + row + _SHIFT
    shifted = jnp.where(g < L, shifted, 0.0)
    out_ref[0, :, 0, :] = cur
    out_ref[0, :, 1, :] = shifted


def kernel(x):
    B, L, D = x.shape
    TL = 1024
    nb = L // TL

    grid = (B, nb)
    out = pl.pallas_call(
        lambda a, b, o: _span_kernel(a, b, o, tl=TL, L=L),
        grid=grid,
        in_specs=[
            pl.BlockSpec((1, TL, D), lambda b, i: (b, i, 0)),
            pl.BlockSpec(
                (1, 16, D),
                lambda b, i: (b, jnp.minimum((i + 1) * (TL // 16), L // 16 - 1), 0),
            ),
        ],
        out_specs=pl.BlockSpec((1, TL, 2, D), lambda b, i: (b, i, 0, 0)),
        out_shape=jax.ShapeDtypeStruct((B, L, 2, D), x.dtype),
    )(x, x)
    return out
